# baseline (speedup 1.0000x reference)
import functools

import jax
import jax.numpy as jnp
from jax import lax
from jax.experimental import pallas as pl
from jax.experimental.pallas import tpu as pltpu
from jax.experimental.pallas import tpu_sc as plsc

NC = 2
NS = 16
NW = NC * NS
LANES = 16
BATCH = 16384
EMBED = 32
BPW = BATCH // NW
GROUPS = BPW // LANES


def _sc_dot_gather(u, i, items, users):
    mesh = plsc.VectorSubcoreMesh(core_axis_name="c", subcore_axis_name="s")

    @functools.partial(
        pl.kernel,
        out_type=jax.ShapeDtypeStruct((BATCH,), jnp.float32),
        mesh=mesh,
        compiler_params=pltpu.CompilerParams(
            needs_layout_passes=False, use_tc_tiling_on_sc=False),
        scratch_types=[
            pltpu.VMEM((BPW,), jnp.int32),
            pltpu.VMEM((BPW,), jnp.int32),
            pltpu.VMEM((BPW, EMBED), jnp.float32),
            pltpu.VMEM((BPW, EMBED), jnp.float32),
            pltpu.VMEM((BPW,), jnp.float32),
            pltpu.SemaphoreType.DMA,
            pltpu.SemaphoreType.DMA,
        ],
    )
    def sc_kernel(u_hbm, i_hbm, users_hbm, items_hbm, out_hbm,
                  uidx_v, iidx_v, urows_v, irows_v, out_v, sem_u, sem_i):
        wid = lax.axis_index("s") * NC + lax.axis_index("c")
        base = wid * BPW
        pltpu.sync_copy(u_hbm.at[pl.ds(base, BPW)], uidx_v)
        pltpu.sync_copy(i_hbm.at[pl.ds(base, BPW)], iidx_v)
        cp_u = pltpu.async_copy(users_hbm.at[uidx_v], urows_v, sem_u)
        cp_i = pltpu.async_copy(items_hbm.at[iidx_v], irows_v, sem_i)
        cp_u.wait()
        cp_i.wait()

        lane = lax.iota(jnp.int32, LANES)

        @pl.loop(0, GROUPS)
        def _(g):
            row0 = g * LANES
            ridx = row0 + lane
            acc = jnp.zeros((LANES,), jnp.float32)
            for d in range(EMBED):
                cidx = jnp.full((LANES,), d, jnp.int32)
                a = plsc.load_gather(irows_v, [ridx, cidx])
                b = plsc.load_gather(urows_v, [ridx, cidx])
                acc = acc + a * b
            out_v[pl.ds(row0, LANES)] = acc

        pltpu.sync_copy(out_v, out_hbm.at[pl.ds(base, BPW)])

    return sc_kernel(u, i, users, items)


def kernel(u, i, items, users):
    return _sc_dot_gather(u, i, items, users)


# speedup vs baseline: 1.0118x; 1.0118x over previous
import functools

import jax
import jax.numpy as jnp
from jax import lax
from jax.experimental import pallas as pl
from jax.experimental.pallas import tpu as pltpu
from jax.experimental.pallas import tpu_sc as plsc

NC = 2
NS = 16
NW = NC * NS
LANES = 16
BATCH = 16384
EMBED = 32
PACK = 128 // EMBED
BPW = BATCH // NW
CH = 128
NCHUNK = BPW // CH
CGROUPS = CH // LANES


def _sc_dot(u, i, items2, users2):
    mesh = plsc.VectorSubcoreMesh(core_axis_name="c", subcore_axis_name="s")

    @functools.partial(
        pl.kernel,
        out_type=jax.ShapeDtypeStruct((BATCH,), jnp.float32),
        mesh=mesh,
        compiler_params=pltpu.CompilerParams(needs_layout_passes=False),
        scratch_types=[
            pltpu.VMEM((BPW,), jnp.int32),
            pltpu.VMEM((BPW,), jnp.int32),
            pltpu.VMEM((CH,), jnp.int32),
            pltpu.VMEM((CH,), jnp.int32),
            pltpu.VMEM((CH,), jnp.int32),
            pltpu.VMEM((CH,), jnp.int32),
            pltpu.VMEM((CH, 128), jnp.float32),
            pltpu.VMEM((CH, 128), jnp.float32),
            pltpu.VMEM((CH, 128), jnp.float32),
            pltpu.VMEM((CH, 128), jnp.float32),
            pltpu.VMEM((BPW,), jnp.float32),
            pltpu.SemaphoreType.DMA,
            pltpu.SemaphoreType.DMA,
            pltpu.SemaphoreType.DMA,
            pltpu.SemaphoreType.DMA,
        ],
    )
    def sc_kernel(u_hbm, i_hbm, users_hbm, items_hbm, out_hbm,
                  uidx_v, iidx_v, urow0, urow1, irow0, irow1,
                  ubuf0, ubuf1, ibuf0, ibuf1, out_v,
                  sem_u0, sem_u1, sem_i0, sem_i1):
        urow = (urow0, urow1)
        irow = (irow0, irow1)
        ubuf = (ubuf0, ubuf1)
        ibuf = (ibuf0, ibuf1)
        sem_u = (sem_u0, sem_u1)
        sem_i = (sem_i0, sem_i1)

        wid = lax.axis_index("s") * NC + lax.axis_index("c")
        base = wid * BPW
        pltpu.sync_copy(u_hbm.at[pl.ds(base, BPW)], uidx_v)
        pltpu.sync_copy(i_hbm.at[pl.ds(base, BPW)], iidx_v)

        lane = lax.iota(jnp.int32, LANES)

        def stage_rows(c):
            p = c % 2

            @pl.loop(0, CGROUPS)
            def _(g):
                s = pl.ds(c * CH + g * LANES, LANES)
                d = pl.ds(g * LANES, LANES)
                urow[p][d] = lax.shift_right_logical(uidx_v[s], 2)
                irow[p][d] = lax.shift_right_logical(iidx_v[s], 2)

            cu = pltpu.async_copy(users_hbm.at[urow[p]], ubuf[p], sem_u[p])
            ci = pltpu.async_copy(items_hbm.at[irow[p]], ibuf[p], sem_i[p])
            return cu, ci

        copies = [None, None]
        copies[0] = stage_rows(0)
        for c in range(NCHUNK):
            p = c % 2
            if c + 1 < NCHUNK:
                copies[(c + 1) % 2] = stage_rows(c + 1)
            cu, ci = copies[p]
            cu.wait()
            ci.wait()

            @pl.loop(0, CGROUPS)
            def _(g, c=c, p=p):
                s = pl.ds(c * CH + g * LANES, LANES)
                row_l = g * LANES + lane
                ucol = (uidx_v[s] & (PACK - 1)) * EMBED
                icol = (iidx_v[s] & (PACK - 1)) * EMBED
                acc = jnp.zeros((LANES,), jnp.float32)
                for d in range(EMBED):
                    skew = (lane + d) & (EMBED - 1)
                    a = plsc.load_gather(ibuf[p], [row_l, icol + skew])
                    b = plsc.load_gather(ubuf[p], [row_l, ucol + skew])
                    acc = acc + a * b
                out_v[pl.ds(c * CH + g * LANES, LANES)] = acc

        pltpu.sync_copy(out_v, out_hbm.at[pl.ds(base, BPW)])

    return sc_kernel(u, i, users2, items2)


def kernel(u, i, items, users):
    items2 = items.reshape(items.shape[0] // PACK, EMBED * PACK)
    users2 = users.reshape(users.shape[0] // PACK, EMBED * PACK)
    return _sc_dot(u, i, items2, users2)


# speedup vs baseline: 3.7608x; 3.7168x over previous
import functools

import jax
import jax.numpy as jnp
from jax import lax
from jax.experimental import pallas as pl
from jax.experimental.pallas import tpu as pltpu
from jax.experimental.pallas import tpu_sc as plsc

NC = 2
NS = 16
NW = NC * NS
LANES = 16
BATCH = 16384
EMBED = 32
BPW = BATCH // NW
SUB = 4
CG = 16
NCG = BPW // CG
NSUB = CG // SUB


def _sc_dot(u, i, items_t, users_t):
    mesh = plsc.VectorSubcoreMesh(core_axis_name="c", subcore_axis_name="s")

    @functools.partial(
        pl.kernel,
        out_type=jax.ShapeDtypeStruct((BATCH,), jnp.float32),
        mesh=mesh,
        compiler_params=pltpu.CompilerParams(needs_layout_passes=False),
        scratch_types=[
            pltpu.VMEM((BPW,), jnp.int32),
            pltpu.VMEM((BPW,), jnp.int32),
            pltpu.VMEM((SUB, EMBED, 128), jnp.float32),
            pltpu.VMEM((SUB, EMBED, 128), jnp.float32),
            pltpu.VMEM((SUB, EMBED, 128), jnp.float32),
            pltpu.VMEM((SUB, EMBED, 128), jnp.float32),
            pltpu.VMEM((NSUB * LANES,), jnp.float32),
            pltpu.VMEM((BPW,), jnp.float32),
            pltpu.SemaphoreType.DMA,
            pltpu.SemaphoreType.DMA,
        ],
    )
    def sc_kernel(u_hbm, i_hbm, ut_hbm, it_hbm, out_hbm,
                  uidx_v, iidx_v, ub0, ub1, ib0, ib1, accbuf, out_v,
                  sem0, sem1):
        ublk = (ub0, ub1)
        iblk = (ib0, ib1)
        sems = (sem0, sem1)

        wid = lax.axis_index("s") * NC + lax.axis_index("c")
        base = wid * BPW
        pltpu.sync_copy(u_hbm.at[pl.ds(base, BPW)], uidx_v)
        pltpu.sync_copy(i_hbm.at[pl.ds(base, BPW)], iidx_v)

        lane = lax.iota(jnp.int32, LANES)
        j_of_lane = lane // SUB
        k_of_lane = lane - j_of_lane * SUB

        def issue(uvec, ivec, jbase, p):
            for j in range(SUB):
                eu = uvec[jbase + j]
                ei = ivec[jbase + j]
                pltpu.async_copy(
                    ut_hbm.at[:, pl.ds((eu >> 7) * 128, 128)],
                    ublk[p].at[j], sems[p])
                pltpu.async_copy(
                    it_hbm.at[:, pl.ds((ei >> 7) * 128, 128)],
                    iblk[p].at[j], sems[p])

        def drain(p):
            for _ in range(2 * SUB):
                pltpu.make_async_copy(
                    ut_hbm.at[:, pl.ds(0, 128)], ublk[p].at[0],
                    sems[p]).wait()

        def lanes_vec(vec, jbase):
            v = jnp.zeros((LANES,), jnp.int32)
            for j in range(SUB):
                v = jnp.where(j_of_lane == j, vec[jbase + j] & 127, v)
            return v

        def extract(uvec, ivec, jbase, p):
            ulane = lanes_vec(uvec, jbase)
            ilane = lanes_vec(ivec, jbase)
            acc = jnp.zeros((LANES,), jnp.float32)
            for m in range(EMBED // SUB):
                c_idx = k_of_lane + (m * SUB)
                a = plsc.load_gather(ublk[p], [j_of_lane, c_idx, ulane])
                b = plsc.load_gather(iblk[p], [j_of_lane, c_idx, ilane])
                acc = acc + a * b
            return acc

        issue(uidx_v[pl.ds(0, LANES)], iidx_v[pl.ds(0, LANES)], 0, 0)

        @pl.loop(0, NCG)
        def _(cg):
            g0 = cg * CG
            uvec = uidx_v[pl.ds(g0, LANES)]
            ivec = iidx_v[pl.ds(g0, LANES)]
            for s in range(NSUB):
                p = s % 2
                if s < NSUB - 1:
                    issue(uvec, ivec, (s + 1) * SUB, 1 - p)
                else:
                    @pl.when(cg < NCG - 1)
                    def _():
                        issue(uidx_v[pl.ds(g0 + CG, LANES)],
                              iidx_v[pl.ds(g0 + CG, LANES)], 0, 1 - p)
                drain(p)
                accbuf[pl.ds(s * LANES, LANES)] = extract(uvec, ivec,
                                                          s * SUB, p)
            out16 = jnp.zeros((LANES,), jnp.float32)
            for k in range(SUB):
                fin = j_of_lane * LANES + k_of_lane * SUB + k
                out16 = out16 + plsc.load_gather(accbuf, [fin])
            out_v[pl.ds(g0, LANES)] = out16

        pltpu.sync_copy(out_v, out_hbm.at[pl.ds(base, BPW)])

    return sc_kernel(u, i, users_t, items_t)


def kernel(u, i, items, users):
    return _sc_dot(u, i, jnp.swapaxes(items, 0, 1), jnp.swapaxes(users, 0, 1))


# speedup vs baseline: 4.0077x; 1.0657x over previous
import functools

import jax
import jax.numpy as jnp
from jax import lax
from jax.experimental import pallas as pl
from jax.experimental.pallas import tpu as pltpu
from jax.experimental.pallas import tpu_sc as plsc

NC = 2
NS = 16
NW = NC * NS
LANES = 16
BATCH = 16384
EMBED = 32
BPW = BATCH // NW
SUB = 4
NSG = BPW // SUB
NCG = BPW // LANES
NBUF = 3
PIPE = NSG - 2


def _sc_dot(u_pad, i_pad, items_t, users_t):
    mesh = plsc.VectorSubcoreMesh(core_axis_name="c", subcore_axis_name="s")

    @functools.partial(
        pl.kernel,
        out_type=jax.ShapeDtypeStruct((BATCH,), jnp.float32),
        mesh=mesh,
        compiler_params=pltpu.CompilerParams(needs_layout_passes=False),
        scratch_types=[
            pltpu.VMEM((BPW + LANES,), jnp.int32),
            pltpu.VMEM((BPW + LANES,), jnp.int32),
            pltpu.VMEM((SUB, EMBED, 128), jnp.float32),
            pltpu.VMEM((SUB, EMBED, 128), jnp.float32),
            pltpu.VMEM((SUB, EMBED, 128), jnp.float32),
            pltpu.VMEM((SUB, EMBED, 128), jnp.float32),
            pltpu.VMEM((SUB, EMBED, 128), jnp.float32),
            pltpu.VMEM((SUB, EMBED, 128), jnp.float32),
            pltpu.VMEM((NSG * LANES,), jnp.float32),
            pltpu.VMEM((BPW,), jnp.float32),
            pltpu.SemaphoreType.DMA,
            pltpu.SemaphoreType.DMA,
            pltpu.SemaphoreType.DMA,
        ],
    )
    def sc_kernel(u_hbm, i_hbm, ut_hbm, it_hbm, out_hbm,
                  uidx_v, iidx_v, ub0, ub1, ub2, ib0, ib1, ib2,
                  accbuf, out_v, sem0, sem1, sem2):
        ublk = (ub0, ub1, ub2)
        iblk = (ib0, ib1, ib2)
        sems = (sem0, sem1, sem2)

        wid = lax.axis_index("s") * NC + lax.axis_index("c")
        base = wid * BPW
        pltpu.sync_copy(u_hbm.at[pl.ds(base, BPW + LANES)], uidx_v)
        pltpu.sync_copy(i_hbm.at[pl.ds(base, BPW + LANES)], iidx_v)

        lane = lax.iota(jnp.int32, LANES)
        j_of_lane = lane // SUB
        k_of_lane = lane - j_of_lane * SUB

        def issue(sg, p):
            uvec = uidx_v[pl.ds(sg * SUB, LANES)]
            ivec = iidx_v[pl.ds(sg * SUB, LANES)]
            for j in range(SUB):
                eu = uvec[j]
                ei = ivec[j]
                pltpu.async_copy(
                    ut_hbm.at[:, pl.ds((eu >> 7) * 128, 128)],
                    ublk[p].at[j], sems[p])
                pltpu.async_copy(
                    it_hbm.at[:, pl.ds((ei >> 7) * 128, 128)],
                    iblk[p].at[j], sems[p])

        def drain(p):
            for _ in range(2 * SUB):
                pltpu.make_async_copy(
                    ut_hbm.at[:, pl.ds(0, 128)], ublk[p].at[0],
                    sems[p]).wait()

        def lanes_vec(vec):
            v = jnp.zeros((LANES,), jnp.int32)
            for j in range(SUB):
                v = jnp.where(j_of_lane == j, vec[j] & 127, v)
            return v

        def extract(sg, p):
            uvec = uidx_v[pl.ds(sg * SUB, LANES)]
            ivec = iidx_v[pl.ds(sg * SUB, LANES)]
            ulane = lanes_vec(uvec)
            ilane = lanes_vec(ivec)
            acc = jnp.zeros((LANES,), jnp.float32)
            for m in range(EMBED // SUB):
                c_idx = k_of_lane + (m * SUB)
                a = plsc.load_gather(ublk[p], [j_of_lane, c_idx, ulane])
                b = plsc.load_gather(iblk[p], [j_of_lane, c_idx, ilane])
                acc = acc + a * b
            accbuf[pl.ds(sg * LANES, LANES)] = acc

        issue(0, 0)
        issue(1, 1)

        @pl.loop(0, PIPE, step=NBUF)
        def _(h):
            for d in range(NBUF):
                sg = h + d
                issue(sg + 2, (d + 2) % NBUF)
                drain(d)
                extract(sg, d)

        for sg, p in ((PIPE, PIPE % NBUF), (PIPE + 1, (PIPE + 1) % NBUF)):
            drain(p)
            extract(sg, p)

        fin = j_of_lane * LANES + k_of_lane * SUB

        @pl.loop(0, NCG)
        def _(cg):
            out16 = jnp.zeros((LANES,), jnp.float32)
            for k in range(SUB):
                out16 = out16 + plsc.load_gather(
                    accbuf, [cg * (SUB * LANES) + fin + k])
            out_v[pl.ds(cg * LANES, LANES)] = out16

        pltpu.sync_copy(out_v, out_hbm.at[pl.ds(base, BPW)])

    return sc_kernel(u_pad, i_pad, users_t, items_t)


def kernel(u, i, items, users):
    u_pad = jnp.pad(u, (0, LANES))
    i_pad = jnp.pad(i, (0, LANES))
    return _sc_dot(u_pad, i_pad,
                   jnp.swapaxes(items, 0, 1), jnp.swapaxes(users, 0, 1))
